# Initial kernel scaffold; baseline (speedup 1.0000x reference)
import jax
import jax.numpy as jnp
from jax.experimental import pallas as pl


def kernel(x):
    raise NotImplementedError("write your pallas kernel here")


# speedup vs baseline: 12.9911x; 12.9911x over previous
import jax
import jax.numpy as jnp
from jax.experimental import pallas as pl

K = 64
ROW_BLOCK = 64


def _topk_mask_kernel(x_ref, o_ref):
    x = x_ref[...]
    b = jax.lax.bitcast_convert_type(x, jnp.uint32)
    key = jnp.where(b >= jnp.uint32(0x80000000), ~b, b | jnp.uint32(0x80000000))
    thr = jnp.zeros((x.shape[0], 1), jnp.uint32)
    for bit in range(31, -1, -1):
        cand = thr | jnp.uint32(1 << bit)
        cnt = jnp.sum((key >= cand).astype(jnp.int32), axis=1, keepdims=True)
        thr = jnp.where(cnt >= K, cand, thr)
    o_ref[...] = jnp.where(key >= thr, x, jnp.float32(0.0))


def kernel(x):
    B, N = x.shape
    return pl.pallas_call(
        _topk_mask_kernel,
        grid=(B // ROW_BLOCK,),
        in_specs=[pl.BlockSpec((ROW_BLOCK, N), lambda i: (i, 0))],
        out_specs=pl.BlockSpec((ROW_BLOCK, N), lambda i: (i, 0)),
        out_shape=jax.ShapeDtypeStruct((B, N), x.dtype),
    )(x)
